# baseline (speedup 1.0000x reference)
import functools

import jax
import jax.numpy as jnp
from jax import lax
from jax.experimental import pallas as pl
from jax.experimental.pallas import tpu as pltpu

_LN_EPS = 1e-5
_LN_TILE = 1024
_Q_TILE = 128
_COL_CHUNK = 2048


def _layernorm_block(x, gamma, beta):
    x = x.astype(jnp.float32)
    mean = jnp.mean(x, axis=-1, keepdims=True)
    centered = x - mean
    var = jnp.mean(centered * centered, axis=-1, keepdims=True)
    nv = centered * lax.rsqrt(var + _LN_EPS)
    return nv * gamma + beta


_NV_DTYPE = jnp.float8_e4m3fn


def _layernorm_kernel(embed_ref, gamma_ref, beta_ref, nodevec_ref):
    nv = _layernorm_block(embed_ref[...], gamma_ref[...], beta_ref[...])
    nodevec_ref[...] = nv.astype(nodevec_ref.dtype)


def _layernorm(embed, gamma, beta):
    n, e = embed.shape
    ln_tile = min(_LN_TILE, n)
    return pl.pallas_call(
        _layernorm_kernel,
        out_shape=jax.ShapeDtypeStruct((n, e), jnp.bfloat16),
        grid=(pl.cdiv(n, ln_tile),),
        in_specs=[
            pl.BlockSpec((ln_tile, e), lambda i: (i, 0)),
            pl.BlockSpec((1, e), lambda i: (0, 0)),
            pl.BlockSpec((1, e), lambda i: (0, 0)),
        ],
        out_specs=pl.BlockSpec((ln_tile, e), lambda i: (i, 0)),
        compiler_params=pltpu.CompilerParams(
            dimension_semantics=("arbitrary",),
        ),
    )(embed, gamma, beta)


def _flash_softmax_rows(q, nv_ref, adj_ref, cw):
    nk = nv_ref.shape[0]
    nchunks = nk // cw
    m = None
    l = None
    chunk_maxes = []
    for c in range(nchunks):
        kc = nv_ref[pl.ds(c * cw, cw), :]
        sc = lax.dot_general(
            q, kc,
            dimension_numbers=(((1,), (1,)), ((), ())),
            preferred_element_type=jnp.float32,
        )
        sc = jnp.maximum(sc, 0.0)
        mc = jnp.max(sc, axis=-1, keepdims=True)
        if m is None:
            m = mc
            pc = jnp.exp(sc - m)
            l = jnp.sum(pc, axis=-1, keepdims=True)
        else:
            m_new = jnp.maximum(m, mc)
            pc = jnp.exp(sc - m_new)
            l = l * jnp.exp(m - m_new) + jnp.sum(pc, axis=-1, keepdims=True)
            m = m_new
        chunk_maxes.append(m)
        adj_ref[:, pl.ds(c * cw, cw)] = pc

    r = pl.reciprocal(l, approx=True)
    for c in range(nchunks):
        factor = jnp.exp(chunk_maxes[c] - m) * r
        sl = pl.ds(c * cw, cw)
        adj_ref[:, sl] = adj_ref[:, sl] * factor


def _fused_kernel(tq, cw, ln_tile, embed_ref, gamma_ref, beta_ref, adj_ref,
                  nv_ref):
    i = pl.program_id(0)
    n = embed_ref.shape[0]

    @pl.when(i == 0)
    def _():
        for t in range(n // ln_tile):
            sl = pl.ds(t * ln_tile, ln_tile)
            nv = _layernorm_block(embed_ref[sl, :], gamma_ref[...],
                                  beta_ref[...])
            nv_ref[sl, :] = nv.astype(nv_ref.dtype)

    q = nv_ref[pl.ds(i * tq, tq), :]
    _flash_softmax_rows(q, nv_ref, adj_ref, cw)


def _adjacency_simple_kernel(q_ref, k_ref, adj_ref):
    scores = lax.dot_general(
        q_ref[...], k_ref[...],
        dimension_numbers=(((1,), (1,)), ((), ())),
        preferred_element_type=jnp.float32,
    )
    s = jnp.maximum(scores, 0.0)
    m = jnp.max(s, axis=-1, keepdims=True)
    p = jnp.exp(s - m)
    denom = jnp.sum(p, axis=-1, keepdims=True)
    adj_ref[...] = p * pl.reciprocal(denom, approx=True)


def kernel(embed, ln_weight, ln_bias):
    num_nodes, embed_dim = embed.shape
    gamma = ln_weight.reshape(1, embed_dim).astype(jnp.float32)
    beta = ln_bias.reshape(1, embed_dim).astype(jnp.float32)

    tq = min(_Q_TILE, num_nodes)
    if (num_nodes % _COL_CHUNK == 0 and num_nodes // _COL_CHUNK >= 2
            and num_nodes % tq == 0 and num_nodes % _LN_TILE == 0):
        return pl.pallas_call(
            functools.partial(_fused_kernel, tq, _COL_CHUNK, _LN_TILE),
            out_shape=jax.ShapeDtypeStruct((num_nodes, num_nodes),
                                           jnp.float32),
            grid=(num_nodes // tq,),
            in_specs=[
                pl.BlockSpec((num_nodes, embed_dim), lambda i: (0, 0)),
                pl.BlockSpec((1, embed_dim), lambda i: (0, 0)),
                pl.BlockSpec((1, embed_dim), lambda i: (0, 0)),
            ],
            out_specs=pl.BlockSpec((tq, num_nodes), lambda i: (i, 0)),
            scratch_shapes=[
                pltpu.VMEM((num_nodes, embed_dim), _NV_DTYPE),
            ],
            compiler_params=pltpu.CompilerParams(
                dimension_semantics=("arbitrary",),
            ),
        )(embed, gamma, beta)

    nodevec = _layernorm(embed, gamma, beta)
    n, e = nodevec.shape
    return pl.pallas_call(
        _adjacency_simple_kernel,
        out_shape=jax.ShapeDtypeStruct((n, n), jnp.float32),
        grid=(pl.cdiv(n, tq),),
        in_specs=[
            pl.BlockSpec((tq, e), lambda i: (i, 0)),
            pl.BlockSpec((n, e), lambda i: (0, 0)),
        ],
        out_specs=pl.BlockSpec((tq, n), lambda i: (i, 0)),
        compiler_params=pltpu.CompilerParams(
            dimension_semantics=("arbitrary",),
        ),
    )(nodevec, nodevec)


# speedup vs baseline: 1.1654x; 1.1654x over previous
import functools

import jax
import jax.numpy as jnp
from jax import lax
from jax.experimental import pallas as pl
from jax.experimental.pallas import tpu as pltpu

_LN_EPS = 1e-5
_LN_TILE = 1024
_Q_TILE = 256
_COL_CHUNK = 2048


def _layernorm_block(x, gamma, beta):
    x = x.astype(jnp.float32)
    mean = jnp.mean(x, axis=-1, keepdims=True)
    centered = x - mean
    var = jnp.mean(centered * centered, axis=-1, keepdims=True)
    nv = centered * lax.rsqrt(var + _LN_EPS)
    return nv * gamma + beta


_NV_DTYPE = jnp.float8_e4m3fn


def _layernorm_kernel(embed_ref, gamma_ref, beta_ref, nodevec_ref):
    nv = _layernorm_block(embed_ref[...], gamma_ref[...], beta_ref[...])
    nodevec_ref[...] = nv.astype(nodevec_ref.dtype)


def _layernorm(embed, gamma, beta):
    n, e = embed.shape
    ln_tile = min(_LN_TILE, n)
    return pl.pallas_call(
        _layernorm_kernel,
        out_shape=jax.ShapeDtypeStruct((n, e), jnp.bfloat16),
        grid=(pl.cdiv(n, ln_tile),),
        in_specs=[
            pl.BlockSpec((ln_tile, e), lambda i: (i, 0)),
            pl.BlockSpec((1, e), lambda i: (0, 0)),
            pl.BlockSpec((1, e), lambda i: (0, 0)),
        ],
        out_specs=pl.BlockSpec((ln_tile, e), lambda i: (i, 0)),
        compiler_params=pltpu.CompilerParams(
            dimension_semantics=("arbitrary",),
        ),
    )(embed, gamma, beta)


def _flash_softmax_rows(q, nv_ref, adj_ref, cw):
    nk = nv_ref.shape[0]
    nchunks = nk // cw
    m = None
    l = None
    chunk_maxes = []
    for c in range(nchunks):
        kc = nv_ref[pl.ds(c * cw, cw), :]
        sc = lax.dot_general(
            q, kc,
            dimension_numbers=(((1,), (1,)), ((), ())),
            preferred_element_type=jnp.float32,
        )
        sc = jnp.maximum(sc, 0.0)
        mc = jnp.max(sc, axis=-1, keepdims=True)
        if m is None:
            m = mc
            pc = jnp.exp(sc - m)
            l = jnp.sum(pc, axis=-1, keepdims=True)
        else:
            m_new = jnp.maximum(m, mc)
            pc = jnp.exp(sc - m_new)
            l = l * jnp.exp(m - m_new) + jnp.sum(pc, axis=-1, keepdims=True)
            m = m_new
        chunk_maxes.append(m)
        adj_ref[:, pl.ds(c * cw, cw)] = pc

    r = pl.reciprocal(l, approx=True)
    for c in range(nchunks):
        factor = jnp.exp(chunk_maxes[c] - m) * r
        sl = pl.ds(c * cw, cw)
        adj_ref[:, sl] = adj_ref[:, sl] * factor


def _fused_kernel(tq, cw, ln_tile, embed_ref, gamma_ref, beta_ref, adj_ref,
                  nv_ref):
    i = pl.program_id(0)
    n = embed_ref.shape[0]

    @pl.when(i == 0)
    def _():
        for t in range(n // ln_tile):
            sl = pl.ds(t * ln_tile, ln_tile)
            nv = _layernorm_block(embed_ref[sl, :], gamma_ref[...],
                                  beta_ref[...])
            nv_ref[sl, :] = nv.astype(nv_ref.dtype)

    q = nv_ref[pl.ds(i * tq, tq), :]
    _flash_softmax_rows(q, nv_ref, adj_ref, cw)


def _adjacency_simple_kernel(q_ref, k_ref, adj_ref):
    scores = lax.dot_general(
        q_ref[...], k_ref[...],
        dimension_numbers=(((1,), (1,)), ((), ())),
        preferred_element_type=jnp.float32,
    )
    s = jnp.maximum(scores, 0.0)
    m = jnp.max(s, axis=-1, keepdims=True)
    p = jnp.exp(s - m)
    denom = jnp.sum(p, axis=-1, keepdims=True)
    adj_ref[...] = p * pl.reciprocal(denom, approx=True)


def kernel(embed, ln_weight, ln_bias):
    num_nodes, embed_dim = embed.shape
    gamma = ln_weight.reshape(1, embed_dim).astype(jnp.float32)
    beta = ln_bias.reshape(1, embed_dim).astype(jnp.float32)

    tq = min(_Q_TILE, num_nodes)
    if (num_nodes % _COL_CHUNK == 0 and num_nodes // _COL_CHUNK >= 2
            and num_nodes % tq == 0 and num_nodes % _LN_TILE == 0):
        return pl.pallas_call(
            functools.partial(_fused_kernel, tq, _COL_CHUNK, _LN_TILE),
            out_shape=jax.ShapeDtypeStruct((num_nodes, num_nodes),
                                           jnp.float32),
            grid=(num_nodes // tq,),
            in_specs=[
                pl.BlockSpec((num_nodes, embed_dim), lambda i: (0, 0)),
                pl.BlockSpec((1, embed_dim), lambda i: (0, 0)),
                pl.BlockSpec((1, embed_dim), lambda i: (0, 0)),
            ],
            out_specs=pl.BlockSpec((tq, num_nodes), lambda i: (i, 0)),
            scratch_shapes=[
                pltpu.VMEM((num_nodes, embed_dim), _NV_DTYPE),
            ],
            compiler_params=pltpu.CompilerParams(
                dimension_semantics=("arbitrary",),
            ),
        )(embed, gamma, beta)

    nodevec = _layernorm(embed, gamma, beta)
    n, e = nodevec.shape
    return pl.pallas_call(
        _adjacency_simple_kernel,
        out_shape=jax.ShapeDtypeStruct((n, n), jnp.float32),
        grid=(pl.cdiv(n, tq),),
        in_specs=[
            pl.BlockSpec((tq, e), lambda i: (i, 0)),
            pl.BlockSpec((n, e), lambda i: (0, 0)),
        ],
        out_specs=pl.BlockSpec((tq, n), lambda i: (i, 0)),
        compiler_params=pltpu.CompilerParams(
            dimension_semantics=("arbitrary",),
        ),
    )(nodevec, nodevec)
